# baseline (speedup 1.0000x reference)
import functools

import jax
import jax.numpy as jnp
from jax import lax
from jax.experimental import pallas as pl
from jax.experimental.pallas import tpu as pltpu
from jax.experimental.pallas import tpu_sc as plsc

N = 10000
DF = 128
E = 320000
NH = 4
DOUT = 128
YW = NH * DOUT

NCORE = 2
NSUB = 16
NTILE = NCORE * NSUB
S = 128
G = 32
NCH = 79
PER_TILE = NCH * S
E_PAD = NTILE * PER_TILE
NPAD = 10112
ROWS_PER_SUB = NPAD // NSUB

_f32 = jnp.float32
_i32 = jnp.int32
_bf16 = jnp.bfloat16

import numpy as _np
_s = _np.arange(YW)
_blk, _off = _s // 32, _s % 32
_LIN_PERM = _np.asarray(
    _blk * 32 + _np.where(_off % 2 == 0, _off // 2, 16 + _off // 2),
    dtype=_np.int32)


def _mm_body(x_ref, wl_ref, wu_ref, ws_ref, y_ref, z_ref, sm_ref):
    xb = x_ref[...]
    dn = (((1,), (1,)), ((), ()))
    y = lax.dot_general(xb, wl_ref[...], dimension_numbers=dn,
                        preferred_element_type=_f32)
    y_ref[...] = y.astype(_bf16)
    z_ref[...] = lax.dot_general(xb, wu_ref[...], dimension_numbers=dn,
                                 preferred_element_type=_f32)
    sm_ref[...] = lax.dot_general(xb, ws_ref[...], dimension_numbers=dn,
                                  preferred_element_type=_f32)


def _mm(x, lin_w_perm, u_w_pad, w_self):
    R = 1000
    return pl.pallas_call(
        _mm_body,
        grid=(N // R,),
        in_specs=[pl.BlockSpec((R, DF), lambda i: (i, 0)),
                  pl.BlockSpec((YW, DF), lambda i: (0, 0)),
                  pl.BlockSpec((DF, DF), lambda i: (0, 0)),
                  pl.BlockSpec((DOUT, DF), lambda i: (0, 0))],
        out_specs=[pl.BlockSpec((R, YW), lambda i: (i, 0)),
                   pl.BlockSpec((R, DF), lambda i: (i, 0)),
                   pl.BlockSpec((R, DOUT), lambda i: (i, 0))],
        out_shape=[jax.ShapeDtypeStruct((N, YW), _bf16),
                   jax.ShapeDtypeStruct((N, DF), _f32),
                   jax.ShapeDtypeStruct((N, DOUT), _f32)],
    )(x, lin_w_perm, u_w_pad, w_self)


def _att_body(with_cnt, *refs):
    if with_cnt:
        (z_hbm, src_hbm, dst_hbm, c_hbm, att_out, cnt_out,
         z_v, src_v, dst_v, attstage, c_v, cnt_v) = refs
    else:
        (z_hbm, src_hbm, dst_hbm, c_hbm, att_out,
         z_v, src_v, dst_v, attstage, c_v) = refs
        cnt_out = cnt_v = None

    cid = lax.axis_index("c")
    sid = lax.axis_index("s")
    wid = sid * NCORE + cid
    pltpu.sync_copy(z_hbm, z_v)
    pltpu.sync_copy(src_hbm.at[wid], src_v)
    pltpu.sync_copy(dst_hbm.at[wid], dst_v)
    pltpu.sync_copy(c_hbm, c_v)

    if with_cnt:
        def _zero_cnt(i, _):
            cnt_v[pl.ds(i * 16, 16)] = jnp.zeros((16,), _f32)
            return 0
        lax.fori_loop(0, NPAD // 16, _zero_cnt, 0)

    cvec = c_v[pl.ds(0, 16)]
    ch = [cvec[h] for h in range(NH)]
    lane = lax.iota(_i32, 16)
    zero16 = jnp.zeros((16,), _i32)

    def chunk_body(j, _):
        for t in range(S // 16):
            o = j * S + t * 16
            s16 = src_v[pl.ds(o, 16)]
            d16 = dst_v[pl.ds(o, 16)]
            zs = [plsc.load_gather(z_v, [s16 + h * N]) for h in range(NH)]
            zd = [plsc.load_gather(z_v, [d16 + h * N]) for h in range(NH)]
            lg = [zd[h] - zs[h] + ch[h] for h in range(NH)]
            m = jnp.maximum(jnp.maximum(lg[0], lg[1]),
                            jnp.maximum(lg[2], lg[3]))
            ex = [jnp.exp(lg[h] - m) for h in range(NH)]
            tot = ex[0] + ex[1] + ex[2] + ex[3]
            w = (s16 != d16).astype(_f32)
            r = w / tot
            eflat = (t * 16 + lane) * NH
            for h in range(NH):
                plsc.store_scatter(attstage, [zero16, eflat + h], ex[h] * r)
            if with_cnt:
                plsc.addupdate_scatter(cnt_v, [d16], w)
        pltpu.sync_copy(attstage, att_out.at[wid * NCH + j])
        return 0

    lax.fori_loop(0, NCH, chunk_body, 0)
    if with_cnt:
        pltpu.sync_copy(cnt_v, cnt_out.at[wid])


def _make_att(with_cnt):
    out_type = [jax.ShapeDtypeStruct((NTILE * NCH, 1, S * NH), _f32)]
    if with_cnt:
        out_type.append(jax.ShapeDtypeStruct((NTILE, NPAD), _f32))
    scratch = [
        pltpu.VMEM((NH * N,), _f32),
        pltpu.VMEM((PER_TILE,), _i32),
        pltpu.VMEM((PER_TILE,), _i32),
        pltpu.VMEM((1, S * NH), _f32),
        pltpu.VMEM((16,), _f32),
    ]
    if with_cnt:
        scratch.append(pltpu.VMEM((NPAD,), _f32))
    return pl.kernel(
        functools.partial(_att_body, with_cnt),
        out_type=tuple(out_type) if with_cnt else out_type[0],
        mesh=plsc.VectorSubcoreMesh(core_axis_name="c", subcore_axis_name="s"),
        scratch_types=scratch,
        compiler_params=pltpu.CompilerParams(needs_layout_passes=False),
    )


_sc_att_cnt = _make_att(True)
_sc_att = _make_att(False)


def _sc_body(y_hbm, att_hbm, src_hbm, dst_hbm, s_out,
             src_v, dst_v, att_v, ybuf, msgbuf, s_sh, sem_y0, sem_y1, sem_sc):
    cid = lax.axis_index("c")
    sid = lax.axis_index("s")
    wid = sid * NCORE + cid

    def _zero_row(i, _):
        for g in range(DOUT // 16):
            msgbuf[0, i, pl.ds(g * 16, 16)] = jnp.zeros((16,), _f32)
        return 0
    lax.fori_loop(0, S, _zero_row, 0)

    base = sid * ROWS_PER_SUB
    off = 0
    while off < ROWS_PER_SUB:
        nrows = min(S, ROWS_PER_SUB - off)
        pltpu.sync_copy(msgbuf.at[0, pl.ds(0, nrows)],
                        s_sh.at[pl.ds(base + off, nrows)])
        off += nrows
    plsc.subcore_barrier()

    sems = (sem_y0, sem_y1)
    mask_hi = jnp.full((16,), -65536, _i32)

    def chunk_body(j, _):
        p = j & 1
        row = wid * NCH + j
        pltpu.sync_copy(src_hbm.at[row], src_v)
        pltpu.sync_copy(dst_hbm.at[row], dst_v.at[pl.ds(p, 1)])
        pltpu.sync_copy(att_hbm.at[row], att_v)

        handles = {0: pltpu.async_copy(
            y_hbm.at[src_v.at[0, pl.ds(0, G)]], ybuf.at[0], sems[0])}
        for g in range(S // G):
            if g + 1 < S // G:
                nb = (g + 1) % 2
                handles[g + 1] = pltpu.async_copy(
                    y_hbm.at[src_v.at[0, pl.ds((g + 1) * G, G)]],
                    ybuf.at[nb], sems[nb])
            handles[g].wait()
            b = g % 2

            for q in range(G // 4):
                av = att_v[0, pl.ds(g * G * NH + q * 16, 16)]
                for rr in range(4):
                    a = [av[4 * rr + h] for h in range(NH)]
                    e = q * 4 + rr
                    for k in range(DOUT // 32):
                        vlo = None
                        vhi = None
                        for h in range(NH):
                            wi = ybuf[b, e, pl.ds(h * (DOUT // 2) + k * 16, 16)]
                            lo = plsc.bitcast(lax.shift_left(wi, 16), _f32)
                            hi = plsc.bitcast(wi & mask_hi, _f32)
                            vlo = a[h] * lo if vlo is None else vlo + a[h] * lo
                            vhi = a[h] * hi if vhi is None else vhi + a[h] * hi
                        mrow = g * G + e
                        msgbuf[p, mrow, pl.ds(k * 32, 16)] = vlo
                        msgbuf[p, mrow, pl.ds(k * 32 + 16, 16)] = vhi

        @pl.when(j >= 1)
        def _drain():
            pltpu.make_async_copy(msgbuf.at[1 - p],
                                  s_sh.at[dst_v.at[1 - p]], sem_sc).wait()
        pltpu.async_copy(msgbuf.at[p], s_sh.at[dst_v.at[p]], sem_sc, add=True)
        return 0

    lax.fori_loop(0, NCH, chunk_body, 0)
    lastp = (NCH - 1) % 2
    pltpu.make_async_copy(msgbuf.at[lastp],
                          s_sh.at[dst_v.at[lastp]], sem_sc).wait()

    plsc.subcore_barrier()
    pltpu.sync_copy(s_sh.at[pl.ds(base, ROWS_PER_SUB)],
                    s_out.at[cid, pl.ds(base, ROWS_PER_SUB)])


_sc_edges = pl.kernel(
    _sc_body,
    out_type=jax.ShapeDtypeStruct((NCORE, NPAD, DOUT), _f32),
    mesh=plsc.VectorSubcoreMesh(core_axis_name="c", subcore_axis_name="s"),
    scratch_types=[
        pltpu.VMEM((1, S), _i32),
        pltpu.VMEM((2, S), _i32),
        pltpu.VMEM((1, S * NH), _f32),
        pltpu.VMEM((2, G, YW // 2), _i32),
        pltpu.VMEM((2, S, DOUT), _f32),
        pltpu.VMEM_SHARED((NPAD, DOUT), _f32),
        pltpu.SemaphoreType.DMA,
        pltpu.SemaphoreType.DMA,
        pltpu.SemaphoreType.DMA,
    ],
    compiler_params=pltpu.CompilerParams(needs_layout_passes=False),
)


def _comb_body(relu, s_ref, inv_ref, sm_ref, b_ref, o_ref):
    s = s_ref[0] + s_ref[1]
    o = (s + sm_ref[...]) * inv_ref[...] + b_ref[...]
    if relu:
        o = jnp.maximum(o, 0.0)
    o_ref[...] = o


def _combine(s_parts, inv, selfm, b, relu):
    R = 1000
    return pl.pallas_call(
        functools.partial(_comb_body, relu),
        grid=(N // R,),
        in_specs=[pl.BlockSpec((NCORE, R, DOUT), lambda i: (0, i, 0)),
                  pl.BlockSpec((R, 1), lambda i: (i, 0)),
                  pl.BlockSpec((R, DOUT), lambda i: (i, 0)),
                  pl.BlockSpec((1, DOUT), lambda i: (0, 0))],
        out_specs=pl.BlockSpec((R, DOUT), lambda i: (i, 0)),
        out_shape=jax.ShapeDtypeStruct((N, DOUT), _f32),
    )(s_parts, inv, selfm, b)


def _layer(h, src2, dst2, src3, dst3, inv, lin_w, u_w, c, b, relu):
    u_pad = jnp.zeros((DF, DF), _f32).at[:NH].set(u_w)
    scw = jax.nn.softmax(c)
    w_self = (scw[:, None, None] * lin_w.reshape(NH, DOUT, DF)).sum(axis=0)
    y, zp, selfm = _mm(h, lin_w[_LIN_PERM], u_pad, w_self)
    y_i32 = lax.bitcast_convert_type(y.reshape(N, YW // 2, 2), _i32)
    z_flat = zp[:, :NH].T.reshape(-1)
    c16 = jnp.zeros((16,), _f32).at[:NH].set(c)
    if inv is None:
        att, cnt_parts = _sc_att_cnt(z_flat, src2, dst2, c16)
        inv = 1.0 / (cnt_parts.sum(axis=0)[:N] + 1.0)
    else:
        att = _sc_att(z_flat, src2, dst2, c16)
    s_parts = _sc_edges(y_i32, att, src3, dst3)
    out = _combine(s_parts, inv[:, None], selfm, b[None, :], relu)
    return out, inv


def kernel(x, edge_index, lin_w1, u_w1, c1, b1, lin_w2, u_w2, c2, b2):
    src = jnp.zeros((E_PAD,), _i32).at[:E].set(edge_index[0])
    dst = jnp.zeros((E_PAD,), _i32).at[:E].set(edge_index[1])
    src3 = src.reshape(NTILE * NCH, 1, S)
    dst3 = dst.reshape(NTILE * NCH, 1, S)
    src2 = src.reshape(NTILE, PER_TILE)
    dst2 = dst.reshape(NTILE, PER_TILE)

    h, inv = _layer(x, src2, dst2, src3, dst3, None,
                    lin_w1, u_w1, c1, b1, True)
    out, _ = _layer(h, src2, dst2, src3, dst3, inv,
                    lin_w2, u_w2, c2, b2, False)
    return out


# speedup vs baseline: 2.4497x; 2.4497x over previous
import functools

import jax
import jax.numpy as jnp
from jax import lax
from jax.experimental import pallas as pl
from jax.experimental.pallas import tpu as pltpu
from jax.experimental.pallas import tpu_sc as plsc

N = 10000
DF = 128
E = 320000
NH = 4
DOUT = 128
YW = NH * DOUT

NCORE = 2
NSUB = 16
NTILE = NCORE * NSUB
S = 128
G = 32
NCH = 79
PER_TILE = NCH * S
E_PAD = NTILE * PER_TILE
NPAD = 10112
ROWS_PER_SUB = NPAD // NSUB

_f32 = jnp.float32
_i32 = jnp.int32
_bf16 = jnp.bfloat16

import numpy as _np
_s = _np.arange(YW)
_blk, _off = _s // 32, _s % 32
_LIN_PERM = _np.asarray(
    _blk * 32 + _np.where(_off % 2 == 0, _off // 2, 16 + _off // 2),
    dtype=_np.int32)


def _mm_body(x_ref, wl_ref, wu_ref, ws_ref, y_ref, z_ref, sm_ref):
    xb = x_ref[...]
    dn = (((1,), (1,)), ((), ()))
    y = lax.dot_general(xb, wl_ref[...], dimension_numbers=dn,
                        preferred_element_type=_f32)
    y_ref[...] = y.astype(_bf16)
    z_ref[...] = lax.dot_general(xb, wu_ref[...], dimension_numbers=dn,
                                 preferred_element_type=_f32)
    sm_ref[...] = lax.dot_general(xb, ws_ref[...], dimension_numbers=dn,
                                  preferred_element_type=_f32)


def _mm(x, lin_w_perm, u_w_pad, w_self):
    R = 1000
    return pl.pallas_call(
        _mm_body,
        grid=(N // R,),
        in_specs=[pl.BlockSpec((R, DF), lambda i: (i, 0)),
                  pl.BlockSpec((YW, DF), lambda i: (0, 0)),
                  pl.BlockSpec((DF, DF), lambda i: (0, 0)),
                  pl.BlockSpec((DOUT, DF), lambda i: (0, 0))],
        out_specs=[pl.BlockSpec((R, YW), lambda i: (i, 0)),
                   pl.BlockSpec((R, DF), lambda i: (i, 0)),
                   pl.BlockSpec((R, DOUT), lambda i: (i, 0))],
        out_shape=[jax.ShapeDtypeStruct((N, YW), _bf16),
                   jax.ShapeDtypeStruct((N, DF), _f32),
                   jax.ShapeDtypeStruct((N, DOUT), _f32)],
    )(x, lin_w_perm, u_w_pad, w_self)


def _att_body(with_cnt, *refs):
    if with_cnt:
        (z_hbm, src_hbm, dst_hbm, c_hbm, att_out, cnt_out,
         z_v, src_v, dst_v, attstage, c_v, cnt_v) = refs
    else:
        (z_hbm, src_hbm, dst_hbm, c_hbm, att_out,
         z_v, src_v, dst_v, attstage, c_v) = refs
        cnt_out = cnt_v = None

    cid = lax.axis_index("c")
    sid = lax.axis_index("s")
    wid = sid * NCORE + cid
    pltpu.sync_copy(z_hbm, z_v)
    pltpu.sync_copy(src_hbm.at[wid], src_v)
    pltpu.sync_copy(dst_hbm.at[wid], dst_v)
    pltpu.sync_copy(c_hbm, c_v)

    if with_cnt:
        def _zero_cnt(i, _):
            cnt_v[pl.ds(i * 16, 16)] = jnp.zeros((16,), _f32)
            return 0
        lax.fori_loop(0, NPAD // 16, _zero_cnt, 0)

    cvec = c_v[pl.ds(0, 16)]
    ch = [cvec[h] for h in range(NH)]
    lane = lax.iota(_i32, 16)
    zero16 = jnp.zeros((16,), _i32)

    def chunk_body(j, _):
        for t in range(S // 16):
            o = j * S + t * 16
            s16 = src_v[pl.ds(o, 16)]
            d16 = dst_v[pl.ds(o, 16)]
            zs = [plsc.load_gather(z_v, [s16 + h * N]) for h in range(NH)]
            zd = [plsc.load_gather(z_v, [d16 + h * N]) for h in range(NH)]
            lg = [zd[h] - zs[h] + ch[h] for h in range(NH)]
            m = jnp.maximum(jnp.maximum(lg[0], lg[1]),
                            jnp.maximum(lg[2], lg[3]))
            ex = [jnp.exp(lg[h] - m) for h in range(NH)]
            tot = ex[0] + ex[1] + ex[2] + ex[3]
            w = (s16 != d16).astype(_f32)
            r = w / tot
            eflat = (t * 16 + lane) * NH
            for h in range(NH):
                plsc.store_scatter(attstage, [zero16, eflat + h], ex[h] * r)
            if with_cnt:
                plsc.addupdate_scatter(cnt_v, [d16], w)
        pltpu.sync_copy(attstage, att_out.at[wid * NCH + j])
        return 0

    lax.fori_loop(0, NCH, chunk_body, 0)
    if with_cnt:
        pltpu.sync_copy(cnt_v, cnt_out.at[wid])


def _make_att(with_cnt):
    out_type = [jax.ShapeDtypeStruct((NTILE * NCH, 1, S * NH), _f32)]
    if with_cnt:
        out_type.append(jax.ShapeDtypeStruct((NTILE, NPAD), _f32))
    scratch = [
        pltpu.VMEM((NH * N,), _f32),
        pltpu.VMEM((PER_TILE,), _i32),
        pltpu.VMEM((PER_TILE,), _i32),
        pltpu.VMEM((1, S * NH), _f32),
        pltpu.VMEM((16,), _f32),
    ]
    if with_cnt:
        scratch.append(pltpu.VMEM((NPAD,), _f32))
    return pl.kernel(
        functools.partial(_att_body, with_cnt),
        out_type=tuple(out_type) if with_cnt else out_type[0],
        mesh=plsc.VectorSubcoreMesh(core_axis_name="c", subcore_axis_name="s"),
        scratch_types=scratch,
        compiler_params=pltpu.CompilerParams(needs_layout_passes=False),
    )


_sc_att_cnt = _make_att(True)
_sc_att = _make_att(False)


def _sc_body(y_hbm, att_hbm, src_hbm, dst_hbm, s_out,
             src_v, dst_v, att_v, ybuf, msgbuf, s_sh, sem_y0, sem_y1, sem_sc):
    cid = lax.axis_index("c")
    sid = lax.axis_index("s")
    wid = sid * NCORE + cid

    def _zero_row(i, _):
        for g in range(DOUT // 16):
            msgbuf[0, i, pl.ds(g * 16, 16)] = jnp.zeros((16,), _f32)
        return 0
    lax.fori_loop(0, S, _zero_row, 0)

    base = sid * ROWS_PER_SUB
    off = 0
    while off < ROWS_PER_SUB:
        nrows = min(S, ROWS_PER_SUB - off)
        pltpu.sync_copy(msgbuf.at[0, pl.ds(0, nrows)],
                        s_sh.at[pl.ds(base + off, nrows)])
        off += nrows
    plsc.subcore_barrier()

    sems = (sem_y0, sem_y1)
    mask_hi = jnp.full((16,), -65536, _i32)

    def chunk_body(j, _):
        p = j & 1
        row = wid * NCH + j
        pltpu.sync_copy(src_hbm.at[row], src_v)
        pltpu.sync_copy(dst_hbm.at[row], dst_v.at[pl.ds(p, 1)])
        pltpu.sync_copy(att_hbm.at[row], att_v)

        handles = {0: pltpu.async_copy(
            y_hbm.at[src_v.at[0, pl.ds(0, G)]], ybuf.at[0], sems[0])}
        for g in range(S // G):
            if g + 1 < S // G:
                nb = (g + 1) % 2
                handles[g + 1] = pltpu.async_copy(
                    y_hbm.at[src_v.at[0, pl.ds((g + 1) * G, G)]],
                    ybuf.at[nb], sems[nb])
            handles[g].wait()
            b = g % 2

            @functools.partial(plsc.parallel_loop, 0, G // 4, unroll=2)
            def quad_body(q):
                av = att_v[0, pl.ds(g * G * NH + q * 16, 16)]
                e0 = q * 4
                for rr in range(4):
                    a = [av[4 * rr + h] for h in range(NH)]
                    e = e0 + rr
                    for k in range(DOUT // 32):
                        vlo = None
                        vhi = None
                        for h in range(NH):
                            wi = ybuf[b, e, pl.ds(h * (DOUT // 2) + k * 16, 16)]
                            lo = plsc.bitcast(lax.shift_left(wi, 16), _f32)
                            hi = plsc.bitcast(wi & mask_hi, _f32)
                            vlo = a[h] * lo if vlo is None else vlo + a[h] * lo
                            vhi = a[h] * hi if vhi is None else vhi + a[h] * hi
                        mrow = g * G + e
                        msgbuf[p, mrow, pl.ds(k * 32, 16)] = vlo
                        msgbuf[p, mrow, pl.ds(k * 32 + 16, 16)] = vhi

        @pl.when(j >= 1)
        def _drain():
            pltpu.make_async_copy(msgbuf.at[1 - p],
                                  s_sh.at[dst_v.at[1 - p]], sem_sc).wait()
        pltpu.async_copy(msgbuf.at[p], s_sh.at[dst_v.at[p]], sem_sc, add=True)
        return 0

    lax.fori_loop(0, NCH, chunk_body, 0)
    lastp = (NCH - 1) % 2
    pltpu.make_async_copy(msgbuf.at[lastp],
                          s_sh.at[dst_v.at[lastp]], sem_sc).wait()

    plsc.subcore_barrier()
    pltpu.sync_copy(s_sh.at[pl.ds(base, ROWS_PER_SUB)],
                    s_out.at[cid, pl.ds(base, ROWS_PER_SUB)])


_sc_edges = pl.kernel(
    _sc_body,
    out_type=jax.ShapeDtypeStruct((NCORE, NPAD, DOUT), _f32),
    mesh=plsc.VectorSubcoreMesh(core_axis_name="c", subcore_axis_name="s"),
    scratch_types=[
        pltpu.VMEM((1, S), _i32),
        pltpu.VMEM((2, S), _i32),
        pltpu.VMEM((1, S * NH), _f32),
        pltpu.VMEM((2, G, YW // 2), _i32),
        pltpu.VMEM((2, S, DOUT), _f32),
        pltpu.VMEM_SHARED((NPAD, DOUT), _f32),
        pltpu.SemaphoreType.DMA,
        pltpu.SemaphoreType.DMA,
        pltpu.SemaphoreType.DMA,
    ],
    compiler_params=pltpu.CompilerParams(needs_layout_passes=False),
)


def _comb_body(relu, s_ref, inv_ref, sm_ref, b_ref, o_ref):
    s = s_ref[0] + s_ref[1]
    o = (s + sm_ref[...]) * inv_ref[...] + b_ref[...]
    if relu:
        o = jnp.maximum(o, 0.0)
    o_ref[...] = o


def _combine(s_parts, inv, selfm, b, relu):
    R = 1000
    return pl.pallas_call(
        functools.partial(_comb_body, relu),
        grid=(N // R,),
        in_specs=[pl.BlockSpec((NCORE, R, DOUT), lambda i: (0, i, 0)),
                  pl.BlockSpec((R, 1), lambda i: (i, 0)),
                  pl.BlockSpec((R, DOUT), lambda i: (i, 0)),
                  pl.BlockSpec((1, DOUT), lambda i: (0, 0))],
        out_specs=pl.BlockSpec((R, DOUT), lambda i: (i, 0)),
        out_shape=jax.ShapeDtypeStruct((N, DOUT), _f32),
    )(s_parts, inv, selfm, b)


def _layer(h, src2, dst2, src3, dst3, inv, lin_w, u_w, c, b, relu):
    u_pad = jnp.zeros((DF, DF), _f32).at[:NH].set(u_w)
    scw = jax.nn.softmax(c)
    w_self = (scw[:, None, None] * lin_w.reshape(NH, DOUT, DF)).sum(axis=0)
    y, zp, selfm = _mm(h, lin_w[_LIN_PERM], u_pad, w_self)
    y_i32 = lax.bitcast_convert_type(y.reshape(N, YW // 2, 2), _i32)
    z_flat = zp[:, :NH].T.reshape(-1)
    c16 = jnp.zeros((16,), _f32).at[:NH].set(c)
    if inv is None:
        att, cnt_parts = _sc_att_cnt(z_flat, src2, dst2, c16)
        inv = 1.0 / (cnt_parts.sum(axis=0)[:N] + 1.0)
    else:
        att = _sc_att(z_flat, src2, dst2, c16)
    s_parts = _sc_edges(y_i32, att, src3, dst3)
    out = _combine(s_parts, inv[:, None], selfm, b[None, :], relu)
    return out, inv


def kernel(x, edge_index, lin_w1, u_w1, c1, b1, lin_w2, u_w2, c2, b2):
    src = jnp.zeros((E_PAD,), _i32).at[:E].set(edge_index[0])
    dst = jnp.zeros((E_PAD,), _i32).at[:E].set(edge_index[1])
    src3 = src.reshape(NTILE * NCH, 1, S)
    dst3 = dst.reshape(NTILE * NCH, 1, S)
    src2 = src.reshape(NTILE, PER_TILE)
    dst2 = dst.reshape(NTILE, PER_TILE)

    h, inv = _layer(x, src2, dst2, src3, dst3, None,
                    lin_w1, u_w1, c1, b1, True)
    out, _ = _layer(h, src2, dst2, src3, dst3, inv,
                    lin_w2, u_w2, c2, b2, False)
    return out
